# baseline (speedup 1.0000x reference)
import jax
import jax.numpy as jnp
from jax import lax
from jax.experimental import pallas as pl
from jax.experimental.pallas import tpu as pltpu
from jax.experimental.pallas import tpu_sc as plsc

BATCH = 16384
FIELDS = 26
FEATURES = 64

NC = 2
NS = 16
NW = NC * NS
CH = 128
NBUF = 4

B = BATCH * FIELDS
BPW = B // NW
NCHUNK = BPW // CH
NGROUP = NCHUNK // NBUF


def _embed_body(idx_hbm, table_hbm, out_hbm, idx_v, rows_v, *sems):
    wid = lax.axis_index("s") * NC + lax.axis_index("c")
    base = wid * BPW

    pltpu.sync_copy(idx_hbm.at[wid], idx_v)

    for b in range(NBUF):
        pltpu.async_copy(table_hbm.at[idx_v.at[b]], rows_v.at[b], sems[b])

    def group(g, carry):
        j = g * NBUF
        for b in range(NBUF):
            chunk = j + b
            pltpu.make_async_copy(
                table_hbm.at[idx_v.at[chunk]], rows_v.at[b], sems[b]
            ).wait()
            pltpu.sync_copy(
                rows_v.at[b], out_hbm.at[pl.ds(base + chunk * CH, CH)]
            )

            @pl.when(chunk + NBUF < NCHUNK)
            def _():
                pltpu.async_copy(
                    table_hbm.at[idx_v.at[chunk + NBUF]], rows_v.at[b], sems[b]
                )

        return carry

    lax.fori_loop(0, NGROUP, group, 0)


@jax.jit
def _run(idx, table):
    f = pl.kernel(
        _embed_body,
        out_type=jax.ShapeDtypeStruct((B, FEATURES), jnp.float32),
        mesh=plsc.VectorSubcoreMesh(core_axis_name="c", subcore_axis_name="s"),
        scratch_types=[
            pltpu.VMEM((NCHUNK, CH), jnp.int32),
            pltpu.VMEM((NBUF, CH, FEATURES), jnp.float32),
        ]
        + [pltpu.SemaphoreType.DMA] * NBUF,
        compiler_params=pltpu.CompilerParams(use_tc_tiling_on_sc=False),
    )
    return f(idx, table)


def kernel(inputs, embedding):
    idx = inputs.astype(jnp.int32).reshape(NW, NCHUNK, CH)
    out = _run(idx, embedding)
    return out.reshape(BATCH, FIELDS, FEATURES)
